# baseline (speedup 1.0000x reference)
import functools

import jax
import jax.numpy as jnp
from jax.experimental import pallas as pl
from jax.experimental.pallas import tpu as pltpu

B, S, D, DS = 4, 2048, 1024, 64
NF, NR, NT = 64, 32, 48
N_ALL = NF + NR + NT
TKF, TKR, TKT = 8, 4, 6


def _topk_mask_normalize(w, k, n):
    wt = jnp.swapaxes(w, 0, 1)
    il = jax.lax.broadcasted_iota(jnp.int32, (1, n), 1)
    jt = jax.lax.broadcasted_iota(jnp.int32, (n, 1), 0)
    beats = (wt > w) | ((wt == w) & (jt < il))
    rank = jnp.sum(beats.astype(jnp.float32), axis=0, keepdims=True)
    sparse = jnp.where(rank < k, w, 0.0)
    return sparse / (jnp.sum(sparse, axis=1, keepdims=True) + 1e-8)


def _router_kernel(x_ref, imp_ref, w_ref, b_ref, ent_ref,
                   of_ref, or_ref, ot_ref):
    h = jnp.dot(x_ref[0], w_ref[...], preferred_element_type=jnp.float32)
    h = h + b_ref[...]
    al = jnp.dot(h, ent_ref[...],
                 preferred_element_type=jnp.float32)

    def seg(lo, n):
        z = al[:, lo:lo + n]
        m = jnp.max(z, axis=1, keepdims=True)
        e = jnp.exp(z - m)
        return e / jnp.sum(e, axis=1, keepdims=True)

    p = jnp.concatenate([seg(0, NF), seg(NF, NR), seg(NF + NR, NT)],
                        axis=1)
    pooled = jnp.dot(imp_ref[0], p,
                     preferred_element_type=jnp.float32)

    of_ref[0] = _topk_mask_normalize(pooled[:, :NF], TKF, NF)
    or_ref[0] = _topk_mask_normalize(pooled[:, NF:NF + NR], TKR, NR)
    ot_ref[0] = _topk_mask_normalize(pooled[:, NF + NR:], TKT, NT)


@functools.partial(jax.jit, static_argnames=("interpret",))
def kernel(x, importance, W_proj, b_proj, neuron_emb, interpret=False):
    imp3 = importance.reshape(B, 1, S)
    b2 = b_proj.reshape(1, DS)
    emb_norm = neuron_emb / (jnp.linalg.norm(neuron_emb, axis=-1,
                                             keepdims=True) + 1e-12)
    ent = emb_norm.T

    of, orr, ot = pl.pallas_call(
        _router_kernel,
        grid=(B,),
        in_specs=[
            pl.BlockSpec((1, S, D), lambda b: (b, 0, 0)),
            pl.BlockSpec((1, 1, S), lambda b: (b, 0, 0)),
            pl.BlockSpec((D, DS), lambda b: (0, 0)),
            pl.BlockSpec((1, DS), lambda b: (0, 0)),
            pl.BlockSpec((DS, N_ALL), lambda b: (0, 0)),
        ],
        out_specs=[
            pl.BlockSpec((1, 1, NF), lambda b: (b, 0, 0)),
            pl.BlockSpec((1, 1, NR), lambda b: (b, 0, 0)),
            pl.BlockSpec((1, 1, NT), lambda b: (b, 0, 0)),
        ],
        out_shape=[
            jax.ShapeDtypeStruct((B, 1, NF), jnp.float32),
            jax.ShapeDtypeStruct((B, 1, NR), jnp.float32),
            jax.ShapeDtypeStruct((B, 1, NT), jnp.float32),
        ],
        compiler_params=pltpu.CompilerParams(
            dimension_semantics=("arbitrary",),
        ),
        interpret=interpret,
    )(x, imp3, W_proj, b2, ent)

    of, orr, ot = of[:, 0], orr[:, 0], ot[:, 0]
    return jnp.concatenate([of, orr, orr, ot], axis=-1)
